# baseline (speedup 1.0000x reference)
import functools

import jax
import jax.numpy as jnp
from jax.experimental import pallas as pl
from jax.experimental.pallas import tpu as pltpu

N = 256
R = 7
D = 768
H = 150
NREL = 6


def _conv_kernel(x_ref, labpad_ref, root_ref, bias_ref, w_ref, out_ref, *, relu):
    r = pl.program_id(0)
    labpad = labpad_ref[...]
    shifted = jnp.concatenate(
        [jnp.full((N, 1), 6, jnp.int32), labpad[:, : N - 1]], axis=1
    )
    ii = jax.lax.broadcasted_iota(jnp.int32, (N, N), 0)
    kk = jax.lax.broadcasted_iota(jnp.int32, (N, N), 1)
    lab_full = jnp.where(kk < ii, labpad, jnp.where(kk > ii, shifted, 6))
    mf = (lab_full == r).astype(jnp.float32)
    m = mf.astype(jnp.bfloat16)
    x = x_ref[...]
    xb = x.astype(jnp.bfloat16)
    sums = jax.lax.dot_general(
        m, xb, (((0,), (0,)), ((), ())), preferred_element_type=jnp.float32
    )
    cnt = jnp.sum(mf, axis=0)
    mean = sums / jnp.maximum(cnt, 1.0)[:, None]
    contrib = jnp.dot(
        mean.astype(jnp.bfloat16), w_ref[0], preferred_element_type=jnp.float32
    )

    @pl.when(r == 0)
    def _init():
        base = jnp.dot(xb, root_ref[...], preferred_element_type=jnp.float32)
        out_ref[...] = base + bias_ref[...] + contrib

    @pl.when(r > 0)
    def _acc():
        out_ref[...] = out_ref[...] + contrib

    if relu:
        @pl.when(r == NREL - 1)
        def _act():
            out_ref[...] = jnp.maximum(out_ref[...], 0.0)


def _rgcn_conv(x, labpad, w, root, bias, relu):
    return pl.pallas_call(
        functools.partial(_conv_kernel, relu=relu),
        grid=(NREL,),
        in_specs=[
            pl.BlockSpec((N, D), lambda r: (0, 0)),
            pl.BlockSpec((N, N), lambda r: (0, 0)),
            pl.BlockSpec((D, D), lambda r: (0, 0)),
            pl.BlockSpec((1, D), lambda r: (0, 0)),
            pl.BlockSpec((1, D, D), lambda r: (r, 0, 0)),
        ],
        out_specs=pl.BlockSpec((N, D), lambda r: (0, 0)),
        out_shape=jax.ShapeDtypeStruct((N, D), jnp.float32),
    )(x, labpad, root, bias, w)


BI = 32


def _pair_kernel(x_ref, w1a_ref, w1b_ref, b1_ref, w2_ref, b2_ref, w3_ref,
                 b3_ref, out_ref, v_ref):
    i = pl.program_id(0)

    @pl.when(i == 0)
    def _precompute_v():
        v_ref[...] = jnp.dot(
            x_ref[...].astype(jnp.bfloat16), w1b_ref[...],
            preferred_element_type=jnp.float32,
        )

    xb = x_ref[pl.ds(i * BI, BI), :].astype(jnp.bfloat16)
    u = jnp.dot(xb, w1a_ref[...], preferred_element_type=jnp.float32)
    v = v_ref[...]
    h1 = jnp.maximum(u[:, None, :] + v[None, :, :] + b1_ref[...][None, :, :], 0.0)
    h1 = h1.reshape(BI * N, H).astype(jnp.bfloat16)
    h2 = jnp.maximum(
        jnp.dot(h1, w2_ref[...], preferred_element_type=jnp.float32) + b2_ref[...],
        0.0,
    )
    s = jnp.dot(
        h2.astype(jnp.bfloat16), w3_ref[...], preferred_element_type=jnp.float32
    ) + b3_ref[...]
    s = s.reshape(BI, N, R)
    ig = i * BI + jax.lax.broadcasted_iota(jnp.int32, (BI, N - 1, 1), 0)
    jj = jax.lax.broadcasted_iota(jnp.int32, (BI, N - 1, 1), 1)
    out_ref[...] = jnp.where(jj < ig, s[:, : N - 1, :], s[:, 1:, :])


def _pair_mlp(x, w1a, w1b, b1, w2, b2, w3, b3):
    return pl.pallas_call(
        _pair_kernel,
        grid=(N // BI,),
        in_specs=[
            pl.BlockSpec((N, D), lambda i: (0, 0)),
            pl.BlockSpec((D, H), lambda i: (0, 0)),
            pl.BlockSpec((D, H), lambda i: (0, 0)),
            pl.BlockSpec((1, H), lambda i: (0, 0)),
            pl.BlockSpec((H, H), lambda i: (0, 0)),
            pl.BlockSpec((1, H), lambda i: (0, 0)),
            pl.BlockSpec((H, R), lambda i: (0, 0)),
            pl.BlockSpec((1, R), lambda i: (0, 0)),
        ],
        out_specs=pl.BlockSpec((BI, N - 1, R), lambda i: (i, 0, 0)),
        out_shape=jax.ShapeDtypeStruct((N, N - 1, R), jnp.float32),
        scratch_shapes=[pltpu.VMEM((N, H), jnp.float32)],
    )(x, w1a, w1b, b1, w2, b2, w3, b3)


def kernel(event_embed, labels, bW1, bb1, bW2, bb2, bW3, bb3,
           cW1, cb1, cW2, cb2, cW3, cb3,
           W1, root1, bias1, W2, root2, bias2):
    x = event_embed[0]
    labpad = jnp.concatenate(
        [labels.reshape(N, N - 1), jnp.full((N, 1), 6, jnp.int32)], axis=1
    )
    bf = jnp.bfloat16
    h = _rgcn_conv(x, labpad, W1.astype(bf), root1.astype(bf),
                   bias1.reshape(1, D), relu=True)
    out = _rgcn_conv(h, labpad, W2.astype(bf), root2.astype(bf),
                     bias2.reshape(1, D), relu=False)
    scores = _pair_mlp(
        out,
        cW1[:D].astype(bf), cW1[D:].astype(bf), cb1.reshape(1, H),
        cW2.astype(bf), cb2.reshape(1, H),
        cW3.astype(bf), cb3.reshape(1, R),
    )
    return scores.reshape(1, N * (N - 1), R)


# speedup vs baseline: 1.2704x; 1.2704x over previous
import jax
import jax.numpy as jnp
from jax.experimental import pallas as pl
from jax.experimental.pallas import tpu as pltpu

N = 256
R = 7
D = 768
H = 150
NREL = 6
BI = 32
STEPS = NREL + NREL + N // BI


def _fused_kernel(x_ref, labpad_ref, root1_ref, bias1_ref, root2_ref,
                  bias2_ref, w1_ref, w2_ref, w1a_ref, w1b_ref, b1_ref,
                  wm2_ref, b2_ref, wm3_ref, b3_ref, out_ref,
                  h_s, o_s, v_s):
    s = pl.program_id(0)
    bf = jnp.bfloat16

    @pl.when(s < 2 * NREL)
    def _conv():
        r = jax.lax.rem(s, NREL)
        phase2 = s >= NREL
        labpad = labpad_ref[...]
        shifted = jnp.concatenate(
            [jnp.full((N, 1), 6, jnp.int32), labpad[:, : N - 1]], axis=1
        )
        ii = jax.lax.broadcasted_iota(jnp.int32, (N, N), 0)
        kk = jax.lax.broadcasted_iota(jnp.int32, (N, N), 1)
        lab_full = jnp.where(kk < ii, labpad, jnp.where(kk > ii, shifted, 6))
        mf = (lab_full == r).astype(jnp.float32)
        m = mf.astype(bf)
        x = jnp.where(phase2, h_s[...], x_ref[...])
        xb = x.astype(bf)
        sums = jax.lax.dot_general(
            m, xb, (((0,), (0,)), ((), ())), preferred_element_type=jnp.float32
        )
        cnt = jnp.sum(mf, axis=0)
        mean = sums / jnp.maximum(cnt, 1.0)[:, None]
        w = jnp.where(phase2, w2_ref[0], w1_ref[0]).astype(bf)
        contrib = jnp.dot(mean.astype(bf), w, preferred_element_type=jnp.float32)

        @pl.when(r == 0)
        def _init():
            root = jnp.where(phase2, root2_ref[...], root1_ref[...]).astype(bf)
            bias = jnp.where(phase2, bias2_ref[...], bias1_ref[...])
            base = jnp.dot(xb, root, preferred_element_type=jnp.float32)
            acc = base + bias + contrib

            @pl.when(phase2)
            def _(): o_s[...] = acc

            @pl.when(jnp.logical_not(phase2))
            def _(): h_s[...] = acc

        @pl.when(r > 0)
        def _acc():
            @pl.when(phase2)
            def _(): o_s[...] = o_s[...] + contrib

            @pl.when(jnp.logical_not(phase2))
            def _(): h_s[...] = h_s[...] + contrib

        @pl.when(jnp.logical_and(r == NREL - 1, jnp.logical_not(phase2)))
        def _act():
            h_s[...] = jnp.maximum(h_s[...], 0.0)

    @pl.when(s >= 2 * NREL)
    def _pair():
        i = s - 2 * NREL

        @pl.when(i == 0)
        def _precompute_v():
            v_s[...] = jnp.dot(
                o_s[...].astype(bf), w1b_ref[...].astype(bf),
                preferred_element_type=jnp.float32,
            )

        xb = o_s[pl.ds(i * BI, BI), :].astype(bf)
        u = jnp.dot(xb, w1a_ref[...].astype(bf),
                    preferred_element_type=jnp.float32)
        v = v_s[...]
        h1 = jnp.maximum(
            u[:, None, :] + v[None, :, :] + b1_ref[...][None, :, :], 0.0
        )
        h1 = h1.reshape(BI * N, H).astype(bf)
        h2 = jnp.maximum(
            jnp.dot(h1, wm2_ref[...].astype(bf),
                    preferred_element_type=jnp.float32) + b2_ref[...],
            0.0,
        )
        sc = jnp.dot(h2.astype(bf), wm3_ref[...].astype(bf),
                     preferred_element_type=jnp.float32) + b3_ref[...]
        sc = sc.reshape(BI, N, R)
        ig = i * BI + jax.lax.broadcasted_iota(jnp.int32, (BI, N - 1, 1), 0)
        jj = jax.lax.broadcasted_iota(jnp.int32, (BI, N - 1, 1), 1)
        out_ref[...] = jnp.where(jj < ig, sc[:, : N - 1, :], sc[:, 1:, :])


def _full(shape):
    return pl.BlockSpec(shape, lambda s: (0,) * len(shape))


def kernel(event_embed, labels, bW1, bb1, bW2, bb2, bW3, bb3,
           cW1, cb1, cW2, cb2, cW3, cb3,
           W1, root1, bias1, W2, root2, bias2):
    x = event_embed[0]
    labpad = jnp.concatenate(
        [labels.reshape(N, N - 1), jnp.full((N, 1), 6, jnp.int32)], axis=1
    )
    scores = pl.pallas_call(
        _fused_kernel,
        grid=(STEPS,),
        in_specs=[
            _full((N, D)),
            _full((N, N)),
            _full((D, D)),
            _full((1, D)),
            _full((D, D)),
            _full((1, D)),
            pl.BlockSpec((1, D, D), lambda s: (jnp.clip(s, 0, NREL - 1), 0, 0)),
            pl.BlockSpec((1, D, D),
                         lambda s: (jnp.clip(s - NREL, 0, NREL - 1), 0, 0)),
            _full((D, H)),
            _full((D, H)),
            _full((1, H)),
            _full((H, H)),
            _full((1, H)),
            _full((H, R)),
            _full((1, R)),
        ],
        out_specs=pl.BlockSpec(
            (BI, N - 1, R), lambda s: (jnp.clip(s - 2 * NREL, 0, N // BI - 1), 0, 0)
        ),
        out_shape=jax.ShapeDtypeStruct((N, N - 1, R), jnp.float32),
        scratch_shapes=[
            pltpu.VMEM((N, D), jnp.float32),
            pltpu.VMEM((N, D), jnp.float32),
            pltpu.VMEM((N, H), jnp.float32),
        ],
    )(x, labpad, root1, bias1.reshape(1, D), root2, bias2.reshape(1, D),
      W1, W2, cW1[:D], cW1[D:], cb1.reshape(1, H), cW2, cb2.reshape(1, H),
      cW3, cb3.reshape(1, R))
    return scores.reshape(1, N * (N - 1), R)


# speedup vs baseline: 1.3038x; 1.0263x over previous
import jax
import jax.numpy as jnp
from jax.experimental import pallas as pl
from jax.experimental.pallas import tpu as pltpu

N = 256
R = 7
D = 768
H = 150
NREL = 6
BI = 64
STEPS = NREL + NREL + N // BI


def _fused_kernel(x_ref, labpadT_ref, root1_ref, bias1_ref, root2_ref,
                  bias2_ref, w1_ref, w2_ref, w1a_ref, w1b_ref, b1_ref,
                  wm2_ref, b2_ref, wm3_ref, b3_ref, out_ref,
                  h_s, o_s, v_s, m_s, mean_s):
    s = pl.program_id(0)
    bf = jnp.bfloat16

    @pl.when(s == 0)
    def _build_masks():
        lt = labpadT_ref[...]
        shifted = jnp.concatenate(
            [jnp.full((1, N), 6, jnp.int32), lt[: N - 1, :]], axis=0
        )
        kk = jax.lax.broadcasted_iota(jnp.int32, (N, N), 0)
        ii = jax.lax.broadcasted_iota(jnp.int32, (N, N), 1)
        labT = jnp.where(kk < ii, lt, jnp.where(kk > ii, shifted, 6))
        for r in range(NREL):
            mr = (labT == r).astype(jnp.float32)
            cnt = jnp.sum(mr, axis=1, keepdims=True)
            m_s[pl.ds(r * N, N), :] = (mr / jnp.maximum(cnt, 1.0)).astype(bf)

    r6 = jax.lax.rem(s, NREL)

    def _conv_step(xb, root_ref, bias_ref, w_ref, acc_ref, first):
        if first:
            mean_s[...] = jnp.dot(
                m_s[...], xb, preferred_element_type=jnp.float32
            ).astype(bf)
        mean_r = mean_s[pl.ds(r6 * N, N), :]
        contrib = jnp.dot(mean_r, w_ref[0].astype(bf),
                          preferred_element_type=jnp.float32)
        if first:
            base = jnp.dot(xb, root_ref[...].astype(bf),
                           preferred_element_type=jnp.float32)
            acc_ref[...] = base + bias_ref[...] + contrib
        else:
            acc_ref[...] = acc_ref[...] + contrib

    @pl.when(s == 0)
    def _c1_first():
        _conv_step(x_ref[...].astype(bf), root1_ref, bias1_ref, w1_ref,
                   h_s, True)

    @pl.when(jnp.logical_and(s < NREL, s > 0))
    def _c1_rest():
        _conv_step(None, root1_ref, bias1_ref, w1_ref, h_s, False)

    @pl.when(s == NREL - 1)
    def _c1_act():
        h_s[...] = jnp.maximum(h_s[...], 0.0)

    @pl.when(s == NREL)
    def _c2_first():
        _conv_step(h_s[...].astype(bf), root2_ref, bias2_ref, w2_ref,
                   o_s, True)

    @pl.when(jnp.logical_and(s < 2 * NREL, s > NREL))
    def _c2_rest():
        _conv_step(None, root2_ref, bias2_ref, w2_ref, o_s, False)

    @pl.when(s >= 2 * NREL)
    def _pair():
        i = s - 2 * NREL

        @pl.when(i == 0)
        def _precompute_v():
            v_s[...] = jnp.dot(
                o_s[...].astype(bf), w1b_ref[...].astype(bf),
                preferred_element_type=jnp.float32,
            ).astype(bf)

        xb = o_s[pl.ds(i * BI, BI), :].astype(bf)
        u = jnp.dot(xb, w1a_ref[...].astype(bf),
                    preferred_element_type=jnp.float32).astype(bf)
        b1 = b1_ref[...].astype(bf)
        h1 = jnp.maximum(u[:, None, :] + v_s[...][None, :, :] + b1[None, :, :],
                         jnp.asarray(0.0, bf))
        h1 = h1.reshape(BI * N, H)
        h2 = jnp.maximum(
            jnp.dot(h1, wm2_ref[...].astype(bf),
                    preferred_element_type=jnp.float32) + b2_ref[...],
            0.0,
        )
        sc = jnp.dot(h2.astype(bf), wm3_ref[...].astype(bf),
                     preferred_element_type=jnp.float32) + b3_ref[...]
        sc = sc.reshape(BI, N, R)
        ig = i * BI + jax.lax.broadcasted_iota(jnp.int32, (BI, N - 1, 1), 0)
        jj = jax.lax.broadcasted_iota(jnp.int32, (BI, N - 1, 1), 1)
        out_ref[...] = jnp.where(jj < ig, sc[:, : N - 1, :], sc[:, 1:, :])


def _full(shape):
    return pl.BlockSpec(shape, lambda s: (0,) * len(shape))


def kernel(event_embed, labels, bW1, bb1, bW2, bb2, bW3, bb3,
           cW1, cb1, cW2, cb2, cW3, cb3,
           W1, root1, bias1, W2, root2, bias2):
    x = event_embed[0]
    labpadT = jnp.concatenate(
        [labels.reshape(N, N - 1), jnp.full((N, 1), 6, jnp.int32)], axis=1
    ).T
    scores = pl.pallas_call(
        _fused_kernel,
        grid=(STEPS,),
        in_specs=[
            _full((N, D)),
            _full((N, N)),
            _full((D, D)),
            _full((1, D)),
            _full((D, D)),
            _full((1, D)),
            pl.BlockSpec((1, D, D), lambda s: (jnp.clip(s, 0, NREL - 1), 0, 0)),
            pl.BlockSpec((1, D, D),
                         lambda s: (jnp.clip(s - NREL, 0, NREL - 1), 0, 0)),
            _full((D, H)),
            _full((D, H)),
            _full((1, H)),
            _full((H, H)),
            _full((1, H)),
            _full((H, R)),
            _full((1, R)),
        ],
        out_specs=pl.BlockSpec(
            (BI, N - 1, R), lambda s: (jnp.clip(s - 2 * NREL, 0, N // BI - 1), 0, 0)
        ),
        out_shape=jax.ShapeDtypeStruct((N, N - 1, R), jnp.float32),
        scratch_shapes=[
            pltpu.VMEM((N, D), jnp.float32),
            pltpu.VMEM((N, D), jnp.float32),
            pltpu.VMEM((N, H), jnp.bfloat16),
            pltpu.VMEM((NREL * N, N), jnp.bfloat16),
            pltpu.VMEM((NREL * N, D), jnp.bfloat16),
        ],
    )(x, labpadT, root1, bias1.reshape(1, D), root2, bias2.reshape(1, D),
      W1, W2, cW1[:D], cW1[D:], cb1.reshape(1, H), cW2, cb2.reshape(1, H),
      cW3, cb3.reshape(1, R))
    return scores.reshape(1, N * (N - 1), R)


# speedup vs baseline: 1.6591x; 1.2725x over previous
import jax
import jax.numpy as jnp
from jax.experimental import pallas as pl
from jax.experimental.pallas import tpu as pltpu

N = 256
R = 7
D = 768
H = 150
NREL = 6
BI = 64
STEPS = NREL + NREL + N // BI


def _fused_kernel(x_ref, labpadT_ref, root1_ref, bias1_ref, root2_ref,
                  bias2_ref, w1_ref, w2_ref, w1a_ref, w1b_ref, b1_ref,
                  wm2_ref, b2_ref, wm3_ref, b3_ref, out_ref,
                  h_s, o_s, v_s, m_s, mean_s):
    s = pl.program_id(0)
    bf = jnp.bfloat16

    @pl.when(s == 0)
    def _build_masks():
        lt = labpadT_ref[...]
        shifted = jnp.concatenate(
            [jnp.full((1, N), 6, jnp.int32), lt[: N - 1, :]], axis=0
        )
        kk = jax.lax.broadcasted_iota(jnp.int32, (N, N), 0)
        ii = jax.lax.broadcasted_iota(jnp.int32, (N, N), 1)
        labT = jnp.where(kk < ii, lt, jnp.where(kk > ii, shifted, 6))
        for r in range(NREL):
            mr = (labT == r).astype(jnp.float32)
            cnt = jnp.sum(mr, axis=1, keepdims=True)
            m_s[pl.ds(r * N, N), :] = (mr / jnp.maximum(cnt, 1.0)).astype(bf)

    r6 = jax.lax.rem(s, NREL)

    def _conv_step(xb, root_ref, bias_ref, w_ref, acc_ref, first):
        if first:
            mean_s[...] = jnp.dot(
                m_s[...], xb, preferred_element_type=jnp.float32
            ).astype(bf)
        mean_r = mean_s[pl.ds(r6 * N, N), :]
        contrib = jnp.dot(mean_r, w_ref[0].astype(bf),
                          preferred_element_type=jnp.float32)
        if first:
            base = jnp.dot(xb, root_ref[...].astype(bf),
                           preferred_element_type=jnp.float32)
            acc_ref[...] = base + bias_ref[...] + contrib
        else:
            acc_ref[...] = acc_ref[...] + contrib

    @pl.when(s == 0)
    def _c1_first():
        _conv_step(x_ref[...].astype(bf), root1_ref, bias1_ref, w1_ref,
                   h_s, True)

    @pl.when(jnp.logical_and(s < NREL, s > 0))
    def _c1_rest():
        _conv_step(None, root1_ref, bias1_ref, w1_ref, h_s, False)

    @pl.when(s == NREL - 1)
    def _c1_act():
        h_s[...] = jnp.maximum(h_s[...], 0.0)

    @pl.when(s == NREL)
    def _c2_first():
        _conv_step(h_s[...].astype(bf), root2_ref, bias2_ref, w2_ref,
                   o_s, True)

    @pl.when(jnp.logical_and(s < 2 * NREL, s > NREL))
    def _c2_rest():
        _conv_step(None, root2_ref, bias2_ref, w2_ref, o_s, False)

    @pl.when(s >= 2 * NREL)
    def _pair():
        i = s - 2 * NREL

        @pl.when(i == 0)
        def _precompute_v():
            v = jnp.dot(
                o_s[...].astype(bf), w1b_ref[...].astype(bf),
                preferred_element_type=jnp.float32,
            ).astype(bf)
            v_s[...] = v.T

        xb = o_s[pl.ds(i * BI, BI), :].astype(bf)
        u = jnp.dot(xb, w1a_ref[...].astype(bf),
                    preferred_element_type=jnp.float32).astype(bf)
        ut = u.T
        b1t = b1_ref[...].astype(bf)
        h1 = jnp.maximum(
            ut[:, :, None] + v_s[...][:, None, :] + b1t[:, :, None],
            jnp.asarray(0.0, bf),
        )
        h1 = h1.reshape(H, BI * N)
        h2 = jnp.maximum(
            jax.lax.dot_general(wm2_ref[...].astype(bf), h1, (((0,), (0,)), ((), ())),
                                preferred_element_type=jnp.float32)
            + b2_ref[...],
            0.0,
        )
        sc = jax.lax.dot_general(wm3_ref[...].astype(bf), h2.astype(bf),
                                 (((0,), (0,)), ((), ())),
                                 preferred_element_type=jnp.float32)
        sc = sc + b3_ref[...]
        sc = sc.reshape(R, BI, N)
        ig = i * BI + jax.lax.broadcasted_iota(jnp.int32, (1, BI, N - 1), 1)
        jj = jax.lax.broadcasted_iota(jnp.int32, (1, BI, N - 1), 2)
        out_ref[...] = jnp.where(jj < ig, sc[:, :, : N - 1], sc[:, :, 1:])


def _full(shape):
    return pl.BlockSpec(shape, lambda s: (0,) * len(shape))


def kernel(event_embed, labels, bW1, bb1, bW2, bb2, bW3, bb3,
           cW1, cb1, cW2, cb2, cW3, cb3,
           W1, root1, bias1, W2, root2, bias2):
    x = event_embed[0]
    labpadT = jnp.concatenate(
        [labels.reshape(N, N - 1), jnp.full((N, 1), 6, jnp.int32)], axis=1
    ).T
    scores = pl.pallas_call(
        _fused_kernel,
        grid=(STEPS,),
        in_specs=[
            _full((N, D)),
            _full((N, N)),
            _full((D, D)),
            _full((1, D)),
            _full((D, D)),
            _full((1, D)),
            pl.BlockSpec((1, D, D), lambda s: (jnp.clip(s, 0, NREL - 1), 0, 0)),
            pl.BlockSpec((1, D, D),
                         lambda s: (jnp.clip(s - NREL, 0, NREL - 1), 0, 0)),
            _full((D, H)),
            _full((D, H)),
            _full((H, 1)),
            _full((H, H)),
            _full((H, 1)),
            _full((H, R)),
            _full((R, 1)),
        ],
        out_specs=pl.BlockSpec(
            (R, BI, N - 1), lambda s: (0, jnp.clip(s - 2 * NREL, 0, N // BI - 1), 0)
        ),
        out_shape=jax.ShapeDtypeStruct((R, N, N - 1), jnp.float32),
        scratch_shapes=[
            pltpu.VMEM((N, D), jnp.float32),
            pltpu.VMEM((N, D), jnp.float32),
            pltpu.VMEM((H, N), jnp.bfloat16),
            pltpu.VMEM((NREL * N, N), jnp.bfloat16),
            pltpu.VMEM((NREL * N, D), jnp.bfloat16),
        ],
    )(x, labpadT, root1, bias1.reshape(1, D), root2, bias2.reshape(1, D),
      W1, W2, cW1[:D], cW1[D:], cb1.reshape(H, 1), cW2, cb2.reshape(H, 1),
      cW3, cb3.reshape(R, 1))
    return scores.transpose(1, 2, 0).reshape(1, N * (N - 1), R)


# speedup vs baseline: 1.8083x; 1.0899x over previous
import jax
import jax.numpy as jnp
from jax.experimental import pallas as pl
from jax.experimental.pallas import tpu as pltpu

N = 256
R = 7
D = 768
H = 150
NREL = 6
WB = 3
CSTEPS = 2 * (NREL // WB)
BI = 64
STEPS = CSTEPS + N // BI


def _fused_kernel(x_ref, labpadT_ref, root1_ref, bias1_ref, root2_ref,
                  bias2_ref, w1_ref, w2_ref, w1a_ref, w1b_ref, b1_ref,
                  wm2_ref, b2_ref, wm3_ref, b3_ref, out_ref,
                  h_s, o_s, v_s, m_s, mean_s):
    s = pl.program_id(0)
    bf = jnp.bfloat16

    @pl.when(s == 0)
    def _build_masks():
        lt = labpadT_ref[...]
        shifted = jnp.concatenate(
            [jnp.full((1, N), 6, jnp.int32), lt[: N - 1, :]], axis=0
        )
        kk = jax.lax.broadcasted_iota(jnp.int32, (N, N), 0)
        ii = jax.lax.broadcasted_iota(jnp.int32, (N, N), 1)
        labT = jnp.where(kk < ii, lt, jnp.where(kk > ii, shifted, 6))
        for r in range(NREL):
            mr = (labT == r).astype(jnp.float32)
            cnt = jnp.sum(mr, axis=1, keepdims=True)
            m_s[pl.ds(r * N, N), :] = (mr / jnp.maximum(cnt, 1.0)).astype(bf)

    half = jax.lax.rem(s, 2)

    def _conv_step(xb, root_ref, bias_ref, w_ref, acc_ref, first):
        if first:
            mean_s[...] = jnp.dot(
                m_s[...], xb, preferred_element_type=jnp.float32
            ).astype(bf)
        rbase = 0 if first else WB
        contrib = None
        for j in range(WB):
            c = jnp.dot(mean_s[pl.ds((rbase + j) * N, N), :],
                        w_ref[j].astype(bf),
                        preferred_element_type=jnp.float32)
            contrib = c if contrib is None else contrib + c
        if first:
            base = jnp.dot(xb, root_ref[...].astype(bf),
                           preferred_element_type=jnp.float32)
            acc_ref[...] = base + bias_ref[...] + contrib
        else:
            acc_ref[...] = acc_ref[...] + contrib

    @pl.when(s == 0)
    def _c1_first():
        _conv_step(x_ref[...].astype(bf), root1_ref, bias1_ref, w1_ref,
                   h_s, True)

    @pl.when(s == 1)
    def _c1_rest():
        _conv_step(None, root1_ref, bias1_ref, w1_ref, h_s, False)
        h_s[...] = jnp.maximum(h_s[...], 0.0)

    @pl.when(s == 2)
    def _c2_first():
        _conv_step(h_s[...].astype(bf), root2_ref, bias2_ref, w2_ref,
                   o_s, True)

    @pl.when(s == 3)
    def _c2_rest():
        _conv_step(None, root2_ref, bias2_ref, w2_ref, o_s, False)

    @pl.when(s >= CSTEPS)
    def _pair():
        i = s - CSTEPS

        @pl.when(i == 0)
        def _precompute_v():
            v = jnp.dot(
                o_s[...].astype(bf), w1b_ref[...].astype(bf),
                preferred_element_type=jnp.float32,
            ).astype(bf)
            v_s[...] = v.T

        xb = o_s[pl.ds(i * BI, BI), :].astype(bf)
        u = jnp.dot(xb, w1a_ref[...].astype(bf),
                    preferred_element_type=jnp.float32).astype(bf)
        ut = u.T
        b1t = b1_ref[...].astype(bf)
        h1 = jnp.maximum(
            ut[:, :, None] + v_s[...][:, None, :] + b1t[:, :, None],
            jnp.asarray(0.0, bf),
        )
        h1 = h1.reshape(H, BI * N)
        h2 = jnp.maximum(
            jax.lax.dot_general(wm2_ref[...].astype(bf), h1, (((0,), (0,)), ((), ())),
                                preferred_element_type=jnp.float32)
            + b2_ref[...],
            0.0,
        )
        sc = jax.lax.dot_general(wm3_ref[...].astype(bf), h2.astype(bf),
                                 (((0,), (0,)), ((), ())),
                                 preferred_element_type=jnp.float32)
        sc = sc + b3_ref[...]
        sc = sc.reshape(R, BI, N)
        ig = i * BI + jax.lax.broadcasted_iota(jnp.int32, (1, BI, N - 1), 1)
        jj = jax.lax.broadcasted_iota(jnp.int32, (1, BI, N - 1), 2)
        out_ref[...] = jnp.where(jj < ig, sc[:, :, : N - 1], sc[:, :, 1:])


def _full(shape):
    return pl.BlockSpec(shape, lambda s: (0,) * len(shape))


def kernel(event_embed, labels, bW1, bb1, bW2, bb2, bW3, bb3,
           cW1, cb1, cW2, cb2, cW3, cb3,
           W1, root1, bias1, W2, root2, bias2):
    x = event_embed[0]
    labpadT = jnp.concatenate(
        [labels.reshape(N, N - 1), jnp.full((N, 1), 6, jnp.int32)], axis=1
    ).T
    scores = pl.pallas_call(
        _fused_kernel,
        grid=(STEPS,),
        in_specs=[
            _full((N, D)),
            _full((N, N)),
            _full((D, D)),
            _full((1, D)),
            _full((D, D)),
            _full((1, D)),
            pl.BlockSpec((WB, D, D), lambda s: (jnp.clip(s, 0, 1), 0, 0)),
            pl.BlockSpec((WB, D, D), lambda s: (jnp.clip(s - 2, 0, 1), 0, 0)),
            _full((D, H)),
            _full((D, H)),
            _full((H, 1)),
            _full((H, H)),
            _full((H, 1)),
            _full((H, R)),
            _full((R, 1)),
        ],
        out_specs=pl.BlockSpec(
            (R, BI, N - 1), lambda s: (0, jnp.clip(s - CSTEPS, 0, N // BI - 1), 0)
        ),
        out_shape=jax.ShapeDtypeStruct((R, N, N - 1), jnp.float32),
        scratch_shapes=[
            pltpu.VMEM((N, D), jnp.float32),
            pltpu.VMEM((N, D), jnp.float32),
            pltpu.VMEM((H, N), jnp.bfloat16),
            pltpu.VMEM((NREL * N, N), jnp.bfloat16),
            pltpu.VMEM((NREL * N, D), jnp.bfloat16),
        ],
    )(x, labpadT, root1, bias1.reshape(1, D), root2, bias2.reshape(1, D),
      W1, W2, cW1[:D], cW1[D:], cb1.reshape(H, 1), cW2, cb2.reshape(H, 1),
      cW3, cb3.reshape(R, 1))
    return scores.transpose(1, 2, 0).reshape(1, N * (N - 1), R)


# speedup vs baseline: 1.8827x; 1.0412x over previous
import jax
import jax.numpy as jnp
from jax.experimental import pallas as pl
from jax.experimental.pallas import tpu as pltpu

N = 256
R = 7
D = 768
H = 150
NREL = 6
WB = 3
CSTEPS = 2 * (NREL // WB)
BI = 64
STEPS = CSTEPS + N // BI


def _fused_kernel(x_ref, labpadT_ref, root1_ref, bias1_ref, root2_ref,
                  bias2_ref, w1_ref, w2_ref, w1_pair_ref, b1_ref,
                  wm2_ref, b2_ref, wm3_ref, b3_ref, out_ref,
                  h_s, o_s, v_s, m_s, mean_s):
    s = pl.program_id(0)
    bf = jnp.bfloat16

    @pl.when(s == 0)
    def _build_masks():
        lt = labpadT_ref[...]
        shifted = jnp.concatenate(
            [jnp.full((1, N), 6, jnp.int32), lt[: N - 1, :]], axis=0
        )
        kk = jax.lax.broadcasted_iota(jnp.int32, (N, N), 0)
        ii = jax.lax.broadcasted_iota(jnp.int32, (N, N), 1)
        labT = jnp.where(kk < ii, lt, jnp.where(kk > ii, shifted, 6))
        for r in range(NREL):
            mr = (labT == r).astype(jnp.float32)
            cnt = jnp.sum(mr, axis=1, keepdims=True)
            m_s[pl.ds(r * N, N), :] = (mr / jnp.maximum(cnt, 1.0)).astype(bf)

    half = jax.lax.rem(s, 2)

    def _conv_step(xb, root_ref, bias_ref, w_ref, acc_ref, first):
        if first:
            mean_s[...] = jnp.dot(
                m_s[...], xb, preferred_element_type=jnp.float32
            ).astype(bf)
        rbase = 0 if first else WB
        contrib = None
        for j in range(WB):
            c = jnp.dot(mean_s[pl.ds((rbase + j) * N, N), :],
                        w_ref[j].astype(bf),
                        preferred_element_type=jnp.float32)
            contrib = c if contrib is None else contrib + c
        if first:
            base = jnp.dot(xb, root_ref[...].astype(bf),
                           preferred_element_type=jnp.float32)
            acc_ref[...] = base + bias_ref[...] + contrib
        else:
            acc_ref[...] = acc_ref[...] + contrib

    @pl.when(s == 0)
    def _c1_first():
        _conv_step(x_ref[...].astype(bf), root1_ref, bias1_ref, w1_ref,
                   h_s, True)

    @pl.when(s == 1)
    def _c1_rest():
        _conv_step(None, root1_ref, bias1_ref, w1_ref, h_s, False)
        h_s[...] = jnp.maximum(h_s[...], 0.0)

    @pl.when(s == 2)
    def _c2_first():
        _conv_step(h_s[...].astype(bf), root2_ref, bias2_ref, w2_ref,
                   o_s, True)

    @pl.when(s == 3)
    def _c2_rest():
        _conv_step(None, root2_ref, bias2_ref, w2_ref, o_s, False)

    @pl.when(s >= CSTEPS)
    def _pair():
        i = s - CSTEPS

        @pl.when(i == 0)
        def _precompute_v():
            v = jnp.dot(
                o_s[...].astype(bf), w1_pair_ref[D:, :].astype(bf),
                preferred_element_type=jnp.float32,
            ).astype(bf)
            v_s[...] = v.T

        xb = o_s[pl.ds(i * BI, BI), :].astype(bf)
        u = jnp.dot(xb, w1_pair_ref[:D, :].astype(bf),
                    preferred_element_type=jnp.float32)
        ut = (u.T + b1_ref[...]).astype(bf)
        h1 = jnp.maximum(
            ut[:, :, None] + v_s[...][:, None, :],
            jnp.asarray(0.0, bf),
        )
        h1 = h1.reshape(H, BI * N)
        h2 = jnp.maximum(
            jax.lax.dot_general(wm2_ref[...].astype(bf), h1, (((0,), (0,)), ((), ())),
                                preferred_element_type=jnp.float32)
            + b2_ref[...],
            0.0,
        )
        sc = jax.lax.dot_general(wm3_ref[...].astype(bf), h2.astype(bf),
                                 (((0,), (0,)), ((), ())),
                                 preferred_element_type=jnp.float32)
        sc = sc + b3_ref[...]
        sc = sc.reshape(R, BI, N)
        ig = i * BI + jax.lax.broadcasted_iota(jnp.int32, (1, BI, N - 1), 1)
        jj = jax.lax.broadcasted_iota(jnp.int32, (1, BI, N - 1), 2)
        out_ref[...] = jnp.where(jj < ig, sc[:, :, : N - 1], sc[:, :, 1:])


def _full(shape):
    return pl.BlockSpec(shape, lambda s: (0,) * len(shape))


def kernel(event_embed, labels, bW1, bb1, bW2, bb2, bW3, bb3,
           cW1, cb1, cW2, cb2, cW3, cb3,
           W1, root1, bias1, W2, root2, bias2):
    x = event_embed[0]
    labpadT = jnp.concatenate(
        [labels.reshape(N, N - 1), jnp.full((N, 1), 6, jnp.int32)], axis=1
    ).T
    scores = pl.pallas_call(
        _fused_kernel,
        grid=(STEPS,),
        in_specs=[
            _full((N, D)),
            _full((N, N)),
            _full((D, D)),
            _full((1, D)),
            _full((D, D)),
            _full((1, D)),
            pl.BlockSpec((WB, D, D), lambda s: (jnp.clip(s, 0, 1), 0, 0)),
            pl.BlockSpec((WB, D, D), lambda s: (jnp.clip(s - 2, 0, 1), 0, 0)),
            _full((2 * D, H)),
            _full((H, 1)),
            _full((H, H)),
            _full((H, 1)),
            _full((H, R)),
            _full((R, 1)),
        ],
        out_specs=pl.BlockSpec(
            (R, BI, N - 1), lambda s: (0, jnp.clip(s - CSTEPS, 0, N // BI - 1), 0)
        ),
        out_shape=jax.ShapeDtypeStruct((R, N, N - 1), jnp.float32),
        scratch_shapes=[
            pltpu.VMEM((N, D), jnp.float32),
            pltpu.VMEM((N, D), jnp.float32),
            pltpu.VMEM((H, N), jnp.bfloat16),
            pltpu.VMEM((NREL * N, N), jnp.bfloat16),
            pltpu.VMEM((NREL * N, D), jnp.bfloat16),
        ],
    )(x, labpadT, root1, bias1.reshape(1, D), root2, bias2.reshape(1, D),
      W1, W2, cW1, cb1.reshape(H, 1), cW2, cb2.reshape(H, 1),
      cW3, cb3.reshape(R, 1))
    return scores.transpose(1, 2, 0).reshape(1, N * (N - 1), R)
